# baseline (speedup 1.0000x reference)
import dataclasses
import functools

import jax
import jax.numpy as jnp
from jax import lax
from jax.experimental import pallas as pl
from jax.experimental.pallas import tpu as pltpu
from jax.experimental.pallas import tpu_sc as plsc

N_NODES = 10000
N_EDGES = 320000
FEATS = 128
HEADS = 8
HC = HEADS * FEATS
XW = HC + FEATS
XWP = 640

NC, NS, L = 2, 16, 16
NW = NC * NS
N_PAD = 10240
DUMMY = N_NODES
E_TOT = N_EDGES + N_NODES
K1 = 64
SUP1 = 1152
NSUP1 = 9
CPS1 = SUP1 // K1
K2 = 16
SUP2 = 576
NSUP2 = 18
CPS2 = SUP2 // K2
EP_TILE = SUP1 * NSUP1
E_PAD = EP_TILE * NW
ROWS_SC = N_PAD // NS

_mesh = plsc.VectorSubcoreMesh(core_axis_name="c", subcore_axis_name="s")

_sc_params = pltpu.CompilerParams()
if "needs_layout_passes" in pltpu.CompilerParams.__dataclass_fields__:
    _sc_params = dataclasses.replace(_sc_params, needs_layout_passes=False)


def _prep_body(x_ref, w_ref, asrc_ref, adst_ref, xh_ref, as_ref, ad_ref):
    xh = jnp.dot(x_ref[...], w_ref[...],
                 preferred_element_type=jnp.float32,
                 precision=lax.Precision.HIGHEST)
    t = xh.reshape(xh.shape[0], HEADS, FEATS)
    a_s = jnp.sum(t * asrc_ref[...], axis=-1)
    a_d = jnp.sum(t * adst_ref[...], axis=-1)
    z = jnp.zeros((xh.shape[0], FEATS - HEADS), jnp.float32)
    xh_ref[...] = jnp.concatenate([xh, a_s, z], axis=1)
    as_ref[...] = jnp.concatenate([a_s, z], axis=1)
    ad_ref[...] = jnp.concatenate([a_d, z], axis=1)


def _tc_prep(x_p, W, a_src, a_dst):
    BLK = 1024
    return pl.pallas_call(
        _prep_body,
        grid=(N_PAD // BLK,),
        in_specs=[pl.BlockSpec((BLK, FEATS), lambda i: (i, 0)),
                  pl.BlockSpec((FEATS, HC), lambda i: (0, 0)),
                  pl.BlockSpec((1, HEADS, FEATS), lambda i: (0, 0, 0)),
                  pl.BlockSpec((1, HEADS, FEATS), lambda i: (0, 0, 0))],
        out_specs=[pl.BlockSpec((BLK, XW), lambda i: (i, 0)),
                   pl.BlockSpec((BLK, FEATS), lambda i: (i, 0)),
                   pl.BlockSpec((BLK, FEATS), lambda i: (i, 0))],
        out_shape=[jax.ShapeDtypeStruct((N_PAD, XW), jnp.float32),
                   jax.ShapeDtypeStruct((N_PAD, FEATS), jnp.float32),
                   jax.ShapeDtypeStruct((N_PAD, FEATS), jnp.float32)],
    )(x_p, W, a_src, a_dst)


def _relu_prep_body(p0_ref, p1_ref, b_ref, w_ref, asrc_ref, adst_ref,
                    h_ref, xh_ref, as_ref, ad_ref):
    h_ref[...] = jnp.maximum(p0_ref[...] + p1_ref[...] + b_ref[...], 0.0)
    _prep_body(h_ref, w_ref, asrc_ref, adst_ref, xh_ref, as_ref, ad_ref)


def _tc_relu_prep(p0, p1, b, W, a_src, a_dst):
    BLK = 1024
    return pl.pallas_call(
        _relu_prep_body,
        grid=(N_PAD // BLK,),
        in_specs=[pl.BlockSpec((BLK, FEATS), lambda i: (i, 0)),
                  pl.BlockSpec((BLK, FEATS), lambda i: (i, 0)),
                  pl.BlockSpec((1, FEATS), lambda i: (0, 0)),
                  pl.BlockSpec((FEATS, HC), lambda i: (0, 0)),
                  pl.BlockSpec((1, HEADS, FEATS), lambda i: (0, 0, 0)),
                  pl.BlockSpec((1, HEADS, FEATS), lambda i: (0, 0, 0))],
        out_specs=[pl.BlockSpec((BLK, FEATS), lambda i: (i, 0)),
                   pl.BlockSpec((BLK, XW), lambda i: (i, 0)),
                   pl.BlockSpec((BLK, FEATS), lambda i: (i, 0)),
                   pl.BlockSpec((BLK, FEATS), lambda i: (i, 0))],
        out_shape=[jax.ShapeDtypeStruct((N_PAD, FEATS), jnp.float32),
                   jax.ShapeDtypeStruct((N_PAD, XW), jnp.float32),
                   jax.ShapeDtypeStruct((N_PAD, FEATS), jnp.float32),
                   jax.ShapeDtypeStruct((N_PAD, FEATS), jnp.float32)],
    )(p0, p1, b, W, a_src, a_dst)


def _btab_body(ad_ref, d0_ref, d1_ref, b_ref):
    den = d0_ref[..., :HEADS] + d1_ref[..., :HEADS]
    inv8 = 1.0 / (den + 1e-16)
    ad8 = ad_ref[..., :HEADS]
    z8 = jnp.zeros_like(ad8)
    zrest = jnp.zeros((ad8.shape[0], FEATS - 3 * HEADS), jnp.float32)
    b_ref[...] = jnp.concatenate([ad8, z8, inv8, zrest], axis=1)


def _tc_btab(ad, d0, d1):
    return pl.pallas_call(
        _btab_body,
        out_shape=jax.ShapeDtypeStruct((N_PAD, FEATS), jnp.float32),
    )(ad, d0, d1)


def _final_body(q0_ref, q1_ref, b_ref, out_ref):
    out_ref[...] = q0_ref[...] + q1_ref[...] + b_ref[...]


def _tc_final(q0, q1, b):
    return pl.pallas_call(
        _final_body,
        out_shape=jax.ShapeDtypeStruct((N_PAD, FEATS), jnp.float32),
    )(q0, q1, b)


def _leaky_exp(a):
    return jnp.exp(jnp.where(a > 0, a, 0.2 * a))


def _zero_accumulator(buf_v, rows, acc_sh, sid):
    @pl.loop(0, rows)
    def _z(j):
        for c in range(FEATS // L):
            buf_v[j, pl.ds(c * L, L)] = jnp.zeros((L,), jnp.float32)

    @pl.loop(0, ROWS_SC // rows)
    def _init(r):
        pltpu.sync_copy(buf_v,
                        acc_sh.at[pl.ds(sid * ROWS_SC + r * rows, rows)])


@functools.partial(
    pl.kernel,
    mesh=_mesh,
    out_type=(jax.ShapeDtypeStruct((N_PAD, FEATS), jnp.float32),
              jax.ShapeDtypeStruct((N_PAD, FEATS), jnp.float32)),
    scratch_types=[pltpu.VMEM((SUP1,), jnp.int32),
                   pltpu.VMEM((SUP1,), jnp.int32),
                   pltpu.VMEM((K1,), jnp.int32),
                   pltpu.VMEM((K1, FEATS), jnp.float32),
                   pltpu.VMEM((K1, FEATS), jnp.float32),
                   pltpu.VMEM((K1, FEATS), jnp.float32),
                   pltpu.VMEM((K1, FEATS), jnp.float32),
                   pltpu.VMEM((K1, FEATS), jnp.float32),
                   pltpu.VMEM_SHARED((N_PAD, FEATS), jnp.float32),
                   pltpu.SemaphoreType.DMA,
                   pltpu.SemaphoreType.DMA,
                   pltpu.SemaphoreType.DMA,
                   pltpu.SemaphoreType.DMA],
)
def _sc_pass1(src_hbm, dst_hbm, as_hbm, ad_hbm,
              d0_hbm, d1_hbm,
              sa_v, da_v, dv, as0_v, ad0_v, as1_v, ad1_v, ex_v, den_sh,
              sema0, semd0, sema1, semd1):
    cid = lax.axis_index("c")
    sid = lax.axis_index("s")
    wid = sid * NC + cid

    _zero_accumulator(ex_v, K1, den_sh, sid)

    @pl.loop(0, K1 // L)
    def _zdv(j):
        dv[pl.ds(j * L, L)] = jnp.zeros((L,), jnp.int32)

    plsc.subcore_barrier()

    base = wid * EP_TILE

    def _issue(c, asb, adb, sema, semd):
        pltpu.async_copy(as_hbm.at[sa_v.at[pl.ds(c * K1, K1)]], asb, sema)
        pltpu.async_copy(ad_hbm.at[da_v.at[pl.ds(c * K1, K1)]], adb, semd)

    def _process(c, asb, adb, sema, semd):
        pltpu.make_async_copy(as_hbm.at[sa_v.at[pl.ds(c * K1, K1)]],
                              asb, sema).wait()
        pltpu.make_async_copy(ad_hbm.at[da_v.at[pl.ds(c * K1, K1)]],
                              adb, semd).wait()

        @plsc.parallel_loop(0, K1, unroll=2)
        def _edge(j):
            ex_v[j, pl.ds(0, L)] = _leaky_exp(asb[j, pl.ds(0, L)]
                                              + adb[j, pl.ds(0, L)])

        @pl.loop(0, K1 // L)
        def _cdv(j):
            dv[pl.ds(j * L, L)] = da_v[pl.ds(c * K1 + j * L, L)]

        pltpu.sync_copy(ex_v, den_sh.at[dv], add=True)

    @pl.loop(0, NSUP1)
    def _super(s):
        soff = base + s * SUP1
        pltpu.sync_copy(src_hbm.at[pl.ds(soff, SUP1)], sa_v)
        pltpu.sync_copy(dst_hbm.at[pl.ds(soff, SUP1)], da_v)
        _issue(0, as0_v, ad0_v, sema0, semd0)

        @pl.loop(0, CPS1 // 2)
        def _pair(p):
            c0 = 2 * p
            _issue(c0 + 1, as1_v, ad1_v, sema1, semd1)
            _process(c0, as0_v, ad0_v, sema0, semd0)

            @pl.when(c0 + 2 < CPS1)
            def _():
                _issue(c0 + 2, as0_v, ad0_v, sema0, semd0)

            _process(c0 + 1, as1_v, ad1_v, sema1, semd1)

    plsc.subcore_barrier()

    stripe = pl.ds(sid * ROWS_SC, ROWS_SC)

    @pl.when(cid == 0)
    def _():
        pltpu.sync_copy(den_sh.at[stripe], d0_hbm.at[stripe])

    @pl.when(cid == 1)
    def _():
        pltpu.sync_copy(den_sh.at[stripe], d1_hbm.at[stripe])


@functools.partial(
    pl.kernel,
    mesh=_mesh,
    compiler_params=_sc_params,
    out_type=(jax.ShapeDtypeStruct((N_PAD, FEATS), jnp.float32),
              jax.ShapeDtypeStruct((N_PAD, FEATS), jnp.float32)),
    scratch_types=[pltpu.VMEM((SUP2,), jnp.int32),
                   pltpu.VMEM((SUP2,), jnp.int32),
                   pltpu.VMEM((K2,), jnp.int32),
                   pltpu.VMEM((K2,), jnp.int32),
                   pltpu.VMEM((K2, XW), jnp.float32),
                   pltpu.VMEM((K2, XW), jnp.float32),
                   pltpu.VMEM((K2, FEATS), jnp.float32),
                   pltpu.VMEM((K2, FEATS), jnp.float32),
                   pltpu.VMEM((K2, FEATS), jnp.float32),
                   pltpu.VMEM((K2, FEATS), jnp.float32),
                   pltpu.VMEM_SHARED((N_PAD, FEATS), jnp.float32),
                   pltpu.SemaphoreType.DMA,
                   pltpu.SemaphoreType.DMA,
                   pltpu.SemaphoreType.DMA,
                   pltpu.SemaphoreType.DMA,
                   pltpu.SemaphoreType.DMA,
                   pltpu.SemaphoreType.DMA],
)
def _sc_pass2(src_hbm, dst_hbm, xh_hbm, b_hbm,
              p0_hbm, p1_hbm,
              sa_v, da_v, dv0, dv1, x0_v, x1_v, b0_v, b1_v, m0_v, m1_v,
              acc_sh,
              semb0, semx0, semb1, semx1, sems0, sems1):
    cid = lax.axis_index("c")
    sid = lax.axis_index("s")
    wid = sid * NC + cid

    _zero_accumulator(m0_v, K2, acc_sh, sid)
    plsc.subcore_barrier()

    base = wid * EP_TILE

    def _issue(c, xbuf, bbuf, semb, semx):
        pltpu.async_copy(b_hbm.at[da_v.at[pl.ds(c * K2, K2)]], bbuf, semb)
        pltpu.async_copy(xh_hbm.at[sa_v.at[pl.ds(c * K2, K2)]], xbuf, semx)

    def _process(s, c, xbuf, bbuf, mbuf, dvbuf, semb, semx, sems):
        pltpu.make_async_copy(b_hbm.at[da_v.at[pl.ds(c * K2, K2)]],
                              bbuf, semb).wait()
        pltpu.make_async_copy(xh_hbm.at[sa_v.at[pl.ds(c * K2, K2)]],
                              xbuf, semx).wait()

        g0 = s * CPS2 + c

        @pl.when(g0 >= 2)
        def _():
            pltpu.make_async_copy(mbuf, acc_sh.at[dvbuf], sems).wait()

        @plsc.parallel_loop(0, K2, unroll=3)
        def _edge(j):
            a = xbuf[j, pl.ds(HC, L)] + bbuf[j, pl.ds(0, L)]
            w_row = _leaky_exp(a) * bbuf[j, pl.ds(L, L)] * (1.0 / HEADS)
            wb = [jnp.broadcast_to(w_row[h], (L,)) for h in range(HEADS)]
            for g in range(FEATS // L):
                acc = wb[0] * xbuf[j, pl.ds(g * L, L)]
                for h in range(1, HEADS):
                    acc = acc + wb[h] * xbuf[j, pl.ds(h * FEATS + g * L, L)]
                mbuf[j, pl.ds(g * L, L)] = acc

        dvbuf[pl.ds(0, K2)] = da_v[pl.ds(c * K2, K2)]
        pltpu.async_copy(mbuf, acc_sh.at[dvbuf], sems, add=True)

    @pl.loop(0, NSUP2)
    def _super(s):
        soff = base + s * SUP2
        pltpu.sync_copy(src_hbm.at[pl.ds(soff, SUP2)], sa_v)
        pltpu.sync_copy(dst_hbm.at[pl.ds(soff, SUP2)], da_v)
        _issue(0, x0_v, b0_v, semb0, semx0)

        @pl.loop(0, CPS2 // 2)
        def _pair(p):
            c0 = 2 * p
            _issue(c0 + 1, x1_v, b1_v, semb1, semx1)
            _process(s, c0, x0_v, b0_v, m0_v, dv0, semb0, semx0, sems0)

            @pl.when(c0 + 2 < CPS2)
            def _():
                _issue(c0 + 2, x0_v, b0_v, semb0, semx0)

            _process(s, c0 + 1, x1_v, b1_v, m1_v, dv1, semb1, semx1, sems1)

    pltpu.make_async_copy(m0_v, acc_sh.at[dv0], sems0).wait()
    pltpu.make_async_copy(m1_v, acc_sh.at[dv1], sems1).wait()
    plsc.subcore_barrier()

    stripe = pl.ds(sid * ROWS_SC, ROWS_SC)

    @pl.when(cid == 0)
    def _():
        pltpu.sync_copy(acc_sh.at[stripe], p0_hbm.at[stripe])

    @pl.when(cid == 1)
    def _():
        pltpu.sync_copy(acc_sh.at[stripe], p1_hbm.at[stripe])


def kernel(x, edge_index, W1, att_src1, att_dst1, b1, W2, att_src2, att_dst2, b2):
    ei = edge_index.astype(jnp.int32)
    loops = jnp.arange(N_NODES, dtype=jnp.int32)
    src = jnp.concatenate([ei[0], loops])
    dst = jnp.concatenate([ei[1], loops])
    src = jnp.pad(src, (0, E_PAD - E_TOT))
    dst = jnp.pad(dst, (0, E_PAD - E_TOT), constant_values=DUMMY)
    x_p = jnp.pad(x, ((0, N_PAD - N_NODES), (0, 0)))
    b1r = b1.reshape(1, FEATS)
    b2r = b2.reshape(1, FEATS)

    xh1, s1, t1 = _tc_prep(x_p, W1, att_src1, att_dst1)
    d0, d1 = _sc_pass1(src, dst, s1, t1)
    btab1 = _tc_btab(t1, d0, d1)
    p0, p1 = _sc_pass2(src, dst, xh1, btab1)
    _, xh2, s2, t2 = _tc_relu_prep(p0, p1, b1r, W2, att_src2, att_dst2)
    e0, e1 = _sc_pass1(src, dst, s2, t2)
    btab2 = _tc_btab(t2, e0, e1)
    q0, q1 = _sc_pass2(src, dst, xh2, btab2)
    out = _tc_final(q0, q1, b2r)
    return out[:N_NODES, :]


# speedup vs baseline: 1.2566x; 1.2566x over previous
import dataclasses
import functools

import jax
import jax.numpy as jnp
from jax import lax
from jax.experimental import pallas as pl
from jax.experimental.pallas import tpu as pltpu
from jax.experimental.pallas import tpu_sc as plsc

N_NODES = 10000
N_EDGES = 320000
FEATS = 128
HEADS = 8
HC = HEADS * FEATS
XW = HC + FEATS
XWP = 640

NC, NS, L = 2, 16, 16
NW = NC * NS
N_PAD = 10240
DUMMY = N_NODES
E_TOT = N_EDGES + N_NODES
K1 = 64
SUP1 = 1152
NSUP1 = 9
CPS1 = SUP1 // K1
K2 = 16
SUP2 = 576
NSUP2 = 18
CPS2 = SUP2 // K2
EP_TILE = SUP1 * NSUP1
E_PAD = EP_TILE * NW
ROWS_SC = N_PAD // NS

_mesh = plsc.VectorSubcoreMesh(core_axis_name="c", subcore_axis_name="s")

_sc_params = pltpu.CompilerParams()
if "needs_layout_passes" in pltpu.CompilerParams.__dataclass_fields__:
    _sc_params = dataclasses.replace(_sc_params, needs_layout_passes=False)


def _prep_body(x_ref, w_ref, asrc_ref, adst_ref, xh_ref, as_ref, ad_ref):
    xh = jnp.dot(x_ref[...], w_ref[...],
                 preferred_element_type=jnp.float32,
                 precision=lax.Precision.HIGHEST)
    t = xh.reshape(xh.shape[0], HEADS, FEATS)
    a_s = jnp.sum(t * asrc_ref[...], axis=-1)
    a_d = jnp.sum(t * adst_ref[...], axis=-1)
    z = jnp.zeros((xh.shape[0], FEATS - HEADS), jnp.float32)
    xh_ref[...] = jnp.concatenate([xh, a_s, z], axis=1)
    as_ref[...] = jnp.concatenate([a_s, z], axis=1)
    ad_ref[...] = jnp.concatenate([a_d, z], axis=1)


def _tc_prep(x_p, W, a_src, a_dst):
    BLK = 1024
    return pl.pallas_call(
        _prep_body,
        grid=(N_PAD // BLK,),
        in_specs=[pl.BlockSpec((BLK, FEATS), lambda i: (i, 0)),
                  pl.BlockSpec((FEATS, HC), lambda i: (0, 0)),
                  pl.BlockSpec((1, HEADS, FEATS), lambda i: (0, 0, 0)),
                  pl.BlockSpec((1, HEADS, FEATS), lambda i: (0, 0, 0))],
        out_specs=[pl.BlockSpec((BLK, XW), lambda i: (i, 0)),
                   pl.BlockSpec((BLK, FEATS), lambda i: (i, 0)),
                   pl.BlockSpec((BLK, FEATS), lambda i: (i, 0))],
        out_shape=[jax.ShapeDtypeStruct((N_PAD, XW), jnp.float32),
                   jax.ShapeDtypeStruct((N_PAD, FEATS), jnp.float32),
                   jax.ShapeDtypeStruct((N_PAD, FEATS), jnp.float32)],
    )(x_p, W, a_src, a_dst)


def _relu_prep_body(p0_ref, p1_ref, b_ref, w_ref, asrc_ref, adst_ref,
                    h_ref, xh_ref, as_ref, ad_ref):
    h_ref[...] = jnp.maximum(p0_ref[...] + p1_ref[...] + b_ref[...], 0.0)
    _prep_body(h_ref, w_ref, asrc_ref, adst_ref, xh_ref, as_ref, ad_ref)


def _tc_relu_prep(p0, p1, b, W, a_src, a_dst):
    BLK = 1024
    return pl.pallas_call(
        _relu_prep_body,
        grid=(N_PAD // BLK,),
        in_specs=[pl.BlockSpec((BLK, FEATS), lambda i: (i, 0)),
                  pl.BlockSpec((BLK, FEATS), lambda i: (i, 0)),
                  pl.BlockSpec((1, FEATS), lambda i: (0, 0)),
                  pl.BlockSpec((FEATS, HC), lambda i: (0, 0)),
                  pl.BlockSpec((1, HEADS, FEATS), lambda i: (0, 0, 0)),
                  pl.BlockSpec((1, HEADS, FEATS), lambda i: (0, 0, 0))],
        out_specs=[pl.BlockSpec((BLK, FEATS), lambda i: (i, 0)),
                   pl.BlockSpec((BLK, XW), lambda i: (i, 0)),
                   pl.BlockSpec((BLK, FEATS), lambda i: (i, 0)),
                   pl.BlockSpec((BLK, FEATS), lambda i: (i, 0))],
        out_shape=[jax.ShapeDtypeStruct((N_PAD, FEATS), jnp.float32),
                   jax.ShapeDtypeStruct((N_PAD, XW), jnp.float32),
                   jax.ShapeDtypeStruct((N_PAD, FEATS), jnp.float32),
                   jax.ShapeDtypeStruct((N_PAD, FEATS), jnp.float32)],
    )(p0, p1, b, W, a_src, a_dst)


def _btab_body(ad_ref, d0_ref, d1_ref, b_ref):
    den = d0_ref[..., :HEADS] + d1_ref[..., :HEADS]
    inv8 = 1.0 / (den + 1e-16)
    ad8 = ad_ref[..., :HEADS]
    z8 = jnp.zeros_like(ad8)
    zrest = jnp.zeros((ad8.shape[0], FEATS - 3 * HEADS), jnp.float32)
    b_ref[...] = jnp.concatenate([ad8, z8, inv8, zrest], axis=1)


def _tc_btab(ad, d0, d1):
    return pl.pallas_call(
        _btab_body,
        out_shape=jax.ShapeDtypeStruct((N_PAD, FEATS), jnp.float32),
    )(ad, d0, d1)


def _final_body(q0_ref, q1_ref, b_ref, out_ref):
    out_ref[...] = q0_ref[...] + q1_ref[...] + b_ref[...]


def _tc_final(q0, q1, b):
    return pl.pallas_call(
        _final_body,
        out_shape=jax.ShapeDtypeStruct((N_PAD, FEATS), jnp.float32),
    )(q0, q1, b)


def _leaky_exp(a):
    return jnp.exp(jnp.where(a > 0, a, 0.2 * a))


def _zero_accumulator(buf_v, rows, acc_sh, sid):
    @pl.loop(0, rows)
    def _z(j):
        for c in range(FEATS // L):
            buf_v[j, pl.ds(c * L, L)] = jnp.zeros((L,), jnp.float32)

    @pl.loop(0, ROWS_SC // rows)
    def _init(r):
        pltpu.sync_copy(buf_v,
                        acc_sh.at[pl.ds(sid * ROWS_SC + r * rows, rows)])


@functools.partial(
    pl.kernel,
    mesh=_mesh,
    out_type=(jax.ShapeDtypeStruct((N_PAD, FEATS), jnp.float32),
              jax.ShapeDtypeStruct((N_PAD, FEATS), jnp.float32)),
    scratch_types=[pltpu.VMEM((SUP1,), jnp.int32),
                   pltpu.VMEM((SUP1,), jnp.int32),
                   pltpu.VMEM((K1,), jnp.int32),
                   pltpu.VMEM((K1, FEATS), jnp.float32),
                   pltpu.VMEM((K1, FEATS), jnp.float32),
                   pltpu.VMEM((K1, FEATS), jnp.float32),
                   pltpu.VMEM((K1, FEATS), jnp.float32),
                   pltpu.VMEM((K1, FEATS), jnp.float32),
                   pltpu.VMEM_SHARED((N_PAD, FEATS), jnp.float32),
                   pltpu.SemaphoreType.DMA,
                   pltpu.SemaphoreType.DMA,
                   pltpu.SemaphoreType.DMA,
                   pltpu.SemaphoreType.DMA],
)
def _sc_pass1(src_hbm, dst_hbm, as_hbm, ad_hbm,
              d0_hbm, d1_hbm,
              sa_v, da_v, dv, as0_v, ad0_v, as1_v, ad1_v, ex_v, den_sh,
              sema0, semd0, sema1, semd1):
    cid = lax.axis_index("c")
    sid = lax.axis_index("s")
    wid = sid * NC + cid

    _zero_accumulator(ex_v, K1, den_sh, sid)

    @pl.loop(0, K1 // L)
    def _zdv(j):
        dv[pl.ds(j * L, L)] = jnp.zeros((L,), jnp.int32)

    plsc.subcore_barrier()

    base = wid * EP_TILE

    def _issue(c, asb, adb, sema, semd):
        pltpu.async_copy(as_hbm.at[sa_v.at[pl.ds(c * K1, K1)]], asb, sema)
        pltpu.async_copy(ad_hbm.at[da_v.at[pl.ds(c * K1, K1)]], adb, semd)

    def _process(c, asb, adb, sema, semd):
        pltpu.make_async_copy(as_hbm.at[sa_v.at[pl.ds(c * K1, K1)]],
                              asb, sema).wait()
        pltpu.make_async_copy(ad_hbm.at[da_v.at[pl.ds(c * K1, K1)]],
                              adb, semd).wait()

        @plsc.parallel_loop(0, K1, unroll=2)
        def _edge(j):
            ex_v[j, pl.ds(0, L)] = _leaky_exp(asb[j, pl.ds(0, L)]
                                              + adb[j, pl.ds(0, L)])

        @pl.loop(0, K1 // L)
        def _cdv(j):
            dv[pl.ds(j * L, L)] = da_v[pl.ds(c * K1 + j * L, L)]

        pltpu.sync_copy(ex_v, den_sh.at[dv], add=True)

    @pl.loop(0, NSUP1)
    def _super(s):
        soff = base + s * SUP1
        pltpu.sync_copy(src_hbm.at[pl.ds(soff, SUP1)], sa_v)
        pltpu.sync_copy(dst_hbm.at[pl.ds(soff, SUP1)], da_v)
        _issue(0, as0_v, ad0_v, sema0, semd0)

        @pl.loop(0, CPS1 // 2)
        def _pair(p):
            c0 = 2 * p
            _issue(c0 + 1, as1_v, ad1_v, sema1, semd1)
            _process(c0, as0_v, ad0_v, sema0, semd0)

            @pl.when(c0 + 2 < CPS1)
            def _():
                _issue(c0 + 2, as0_v, ad0_v, sema0, semd0)

            _process(c0 + 1, as1_v, ad1_v, sema1, semd1)

    plsc.subcore_barrier()

    stripe = pl.ds(sid * ROWS_SC, ROWS_SC)

    @pl.when(cid == 0)
    def _():
        pltpu.sync_copy(den_sh.at[stripe], d0_hbm.at[stripe])

    @pl.when(cid == 1)
    def _():
        pltpu.sync_copy(den_sh.at[stripe], d1_hbm.at[stripe])


@functools.partial(
    pl.kernel,
    mesh=_mesh,
    compiler_params=_sc_params,
    out_type=(jax.ShapeDtypeStruct((N_PAD, FEATS), jnp.float32),
              jax.ShapeDtypeStruct((N_PAD, FEATS), jnp.float32)),
    scratch_types=[pltpu.VMEM((SUP2,), jnp.int32),
                   pltpu.VMEM((SUP2,), jnp.int32),
                   pltpu.VMEM((K2,), jnp.int32),
                   pltpu.VMEM((K2,), jnp.int32),
                   pltpu.VMEM((K2, XW), jnp.float32),
                   pltpu.VMEM((K2, XW), jnp.float32),
                   pltpu.VMEM((K2, FEATS), jnp.float32),
                   pltpu.VMEM((K2, FEATS), jnp.float32),
                   pltpu.VMEM((K2, FEATS), jnp.float32),
                   pltpu.VMEM((K2, FEATS), jnp.float32),
                   pltpu.VMEM_SHARED((N_PAD, FEATS), jnp.float32),
                   pltpu.SemaphoreType.DMA,
                   pltpu.SemaphoreType.DMA,
                   pltpu.SemaphoreType.DMA,
                   pltpu.SemaphoreType.DMA,
                   pltpu.SemaphoreType.DMA,
                   pltpu.SemaphoreType.DMA],
)
def _sc_pass2(src_hbm, dst_hbm, xh_hbm, b_hbm,
              p0_hbm, p1_hbm,
              sa_v, da_v, dv0, dv1, x0_v, x1_v, b0_v, b1_v, m0_v, m1_v,
              acc_sh,
              semb0, semx0, semb1, semx1, sems0, sems1):
    cid = lax.axis_index("c")
    sid = lax.axis_index("s")
    wid = sid * NC + cid

    _zero_accumulator(m0_v, K2, acc_sh, sid)
    plsc.subcore_barrier()

    base = wid * EP_TILE

    def _issue(c, xbuf, bbuf, semb, semx):
        pltpu.async_copy(b_hbm.at[da_v.at[pl.ds(c * K2, K2)]], bbuf, semb)
        pltpu.async_copy(xh_hbm.at[sa_v.at[pl.ds(c * K2, K2)]], xbuf, semx)

    def _process(s, c, xbuf, bbuf, mbuf, dvbuf, semb, semx, sems):
        pltpu.make_async_copy(b_hbm.at[da_v.at[pl.ds(c * K2, K2)]],
                              bbuf, semb).wait()
        pltpu.make_async_copy(xh_hbm.at[sa_v.at[pl.ds(c * K2, K2)]],
                              xbuf, semx).wait()

        g0 = s * CPS2 + c

        @pl.when(g0 >= 2)
        def _():
            pltpu.make_async_copy(mbuf, acc_sh.at[dvbuf], sems).wait()

        @plsc.parallel_loop(0, K2, unroll=2)
        def _edge(j):
            a = xbuf[j, pl.ds(HC, L)] + bbuf[j, pl.ds(0, L)]
            w_row = _leaky_exp(a) * bbuf[j, pl.ds(L, L)] * (1.0 / HEADS)
            wb = [jnp.broadcast_to(w_row[h], (L,)) for h in range(HEADS)]
            for g in range(FEATS // L):
                acc = wb[0] * xbuf[j, pl.ds(g * L, L)]
                for h in range(1, HEADS):
                    acc = acc + wb[h] * xbuf[j, pl.ds(h * FEATS + g * L, L)]
                mbuf[j, pl.ds(g * L, L)] = acc

        dvbuf[pl.ds(0, K2)] = da_v[pl.ds(c * K2, K2)]
        pltpu.async_copy(mbuf, acc_sh.at[dvbuf], sems, add=True)

    @pl.loop(0, NSUP2)
    def _super(s):
        soff = base + s * SUP2
        pltpu.sync_copy(src_hbm.at[pl.ds(soff, SUP2)], sa_v)
        pltpu.sync_copy(dst_hbm.at[pl.ds(soff, SUP2)], da_v)
        _issue(0, x0_v, b0_v, semb0, semx0)

        @pl.loop(0, CPS2 // 2)
        def _pair(p):
            c0 = 2 * p
            _issue(c0 + 1, x1_v, b1_v, semb1, semx1)
            _process(s, c0, x0_v, b0_v, m0_v, dv0, semb0, semx0, sems0)

            @pl.when(c0 + 2 < CPS2)
            def _():
                _issue(c0 + 2, x0_v, b0_v, semb0, semx0)

            _process(s, c0 + 1, x1_v, b1_v, m1_v, dv1, semb1, semx1, sems1)

    pltpu.make_async_copy(m0_v, acc_sh.at[dv0], sems0).wait()
    pltpu.make_async_copy(m1_v, acc_sh.at[dv1], sems1).wait()
    plsc.subcore_barrier()

    stripe = pl.ds(sid * ROWS_SC, ROWS_SC)

    @pl.when(cid == 0)
    def _():
        pltpu.sync_copy(acc_sh.at[stripe], p0_hbm.at[stripe])

    @pl.when(cid == 1)
    def _():
        pltpu.sync_copy(acc_sh.at[stripe], p1_hbm.at[stripe])


def kernel(x, edge_index, W1, att_src1, att_dst1, b1, W2, att_src2, att_dst2, b2):
    ei = edge_index.astype(jnp.int32)
    loops = jnp.arange(N_NODES, dtype=jnp.int32)
    src = jnp.concatenate([ei[0], loops])
    dst = jnp.concatenate([ei[1], loops])
    src = jnp.pad(src, (0, E_PAD - E_TOT))
    dst = jnp.pad(dst, (0, E_PAD - E_TOT), constant_values=DUMMY)
    x_p = jnp.pad(x, ((0, N_PAD - N_NODES), (0, 0)))
    b1r = b1.reshape(1, FEATS)
    b2r = b2.reshape(1, FEATS)

    xh1, s1, t1 = _tc_prep(x_p, W1, att_src1, att_dst1)
    d0, d1 = _sc_pass1(src, dst, s1, t1)
    btab1 = _tc_btab(t1, d0, d1)
    p0, p1 = _sc_pass2(src, dst, xh1, btab1)
    _, xh2, s2, t2 = _tc_relu_prep(p0, p1, b1r, W2, att_src2, att_dst2)
    e0, e1 = _sc_pass1(src, dst, s2, t2)
    btab2 = _tc_btab(t2, e0, e1)
    q0, q1 = _sc_pass2(src, dst, xh2, btab2)
    out = _tc_final(q0, q1, b2r)
    return out[:N_NODES, :]


# speedup vs baseline: 1.2734x; 1.0134x over previous
import dataclasses
import functools

import jax
import jax.numpy as jnp
from jax import lax
from jax.experimental import pallas as pl
from jax.experimental.pallas import tpu as pltpu
from jax.experimental.pallas import tpu_sc as plsc

N_NODES = 10000
N_EDGES = 320000
FEATS = 128
HEADS = 8
HC = HEADS * FEATS
XW = HC + FEATS
XWP = 640

NC, NS, L = 2, 16, 16
NW = NC * NS
N_PAD = 10240
DUMMY = N_NODES
E_TOT = N_EDGES + N_NODES
K1 = 48
SUP1 = 864
NSUP1 = 12
CPS1 = SUP1 // K1
K2 = 16
SUP2 = 1152
NSUP2 = 9
CPS2 = SUP2 // K2
EP_TILE = SUP1 * NSUP1
E_PAD = EP_TILE * NW
ROWS_SC = N_PAD // NS

_mesh = plsc.VectorSubcoreMesh(core_axis_name="c", subcore_axis_name="s")

_sc_params = pltpu.CompilerParams()
if "needs_layout_passes" in pltpu.CompilerParams.__dataclass_fields__:
    _sc_params = dataclasses.replace(_sc_params, needs_layout_passes=False)


def _prep_body(x_ref, w_ref, asrc_ref, adst_ref, xh_ref, as_ref, ad_ref):
    xh = jnp.dot(x_ref[...], w_ref[...],
                 preferred_element_type=jnp.float32,
                 precision=lax.Precision.HIGHEST)
    t = xh.reshape(xh.shape[0], HEADS, FEATS)
    a_s = jnp.sum(t * asrc_ref[...], axis=-1)
    a_d = jnp.sum(t * adst_ref[...], axis=-1)
    z = jnp.zeros((xh.shape[0], FEATS - HEADS), jnp.float32)
    xh_ref[...] = jnp.concatenate([xh, a_s, z], axis=1)
    as_ref[...] = jnp.concatenate([a_s, z], axis=1)
    ad_ref[...] = jnp.concatenate([a_d, z], axis=1)


def _tc_prep(x_p, W, a_src, a_dst):
    BLK = 1024
    return pl.pallas_call(
        _prep_body,
        grid=(N_PAD // BLK,),
        in_specs=[pl.BlockSpec((BLK, FEATS), lambda i: (i, 0)),
                  pl.BlockSpec((FEATS, HC), lambda i: (0, 0)),
                  pl.BlockSpec((1, HEADS, FEATS), lambda i: (0, 0, 0)),
                  pl.BlockSpec((1, HEADS, FEATS), lambda i: (0, 0, 0))],
        out_specs=[pl.BlockSpec((BLK, XW), lambda i: (i, 0)),
                   pl.BlockSpec((BLK, FEATS), lambda i: (i, 0)),
                   pl.BlockSpec((BLK, FEATS), lambda i: (i, 0))],
        out_shape=[jax.ShapeDtypeStruct((N_PAD, XW), jnp.float32),
                   jax.ShapeDtypeStruct((N_PAD, FEATS), jnp.float32),
                   jax.ShapeDtypeStruct((N_PAD, FEATS), jnp.float32)],
    )(x_p, W, a_src, a_dst)


def _relu_prep_body(p0_ref, p1_ref, b_ref, w_ref, asrc_ref, adst_ref,
                    h_ref, xh_ref, as_ref, ad_ref):
    h_ref[...] = jnp.maximum(p0_ref[...] + p1_ref[...] + b_ref[...], 0.0)
    _prep_body(h_ref, w_ref, asrc_ref, adst_ref, xh_ref, as_ref, ad_ref)


def _tc_relu_prep(p0, p1, b, W, a_src, a_dst):
    BLK = 1024
    return pl.pallas_call(
        _relu_prep_body,
        grid=(N_PAD // BLK,),
        in_specs=[pl.BlockSpec((BLK, FEATS), lambda i: (i, 0)),
                  pl.BlockSpec((BLK, FEATS), lambda i: (i, 0)),
                  pl.BlockSpec((1, FEATS), lambda i: (0, 0)),
                  pl.BlockSpec((FEATS, HC), lambda i: (0, 0)),
                  pl.BlockSpec((1, HEADS, FEATS), lambda i: (0, 0, 0)),
                  pl.BlockSpec((1, HEADS, FEATS), lambda i: (0, 0, 0))],
        out_specs=[pl.BlockSpec((BLK, FEATS), lambda i: (i, 0)),
                   pl.BlockSpec((BLK, XW), lambda i: (i, 0)),
                   pl.BlockSpec((BLK, FEATS), lambda i: (i, 0)),
                   pl.BlockSpec((BLK, FEATS), lambda i: (i, 0))],
        out_shape=[jax.ShapeDtypeStruct((N_PAD, FEATS), jnp.float32),
                   jax.ShapeDtypeStruct((N_PAD, XW), jnp.float32),
                   jax.ShapeDtypeStruct((N_PAD, FEATS), jnp.float32),
                   jax.ShapeDtypeStruct((N_PAD, FEATS), jnp.float32)],
    )(p0, p1, b, W, a_src, a_dst)


def _btab_body(ad_ref, d0_ref, d1_ref, b_ref):
    den = d0_ref[..., :HEADS] + d1_ref[..., :HEADS]
    inv8 = 1.0 / (den + 1e-16)
    ad8 = ad_ref[..., :HEADS]
    z8 = jnp.zeros_like(ad8)
    zrest = jnp.zeros((ad8.shape[0], FEATS - 3 * HEADS), jnp.float32)
    b_ref[...] = jnp.concatenate([ad8, z8, inv8, zrest], axis=1)


def _tc_btab(ad, d0, d1):
    return pl.pallas_call(
        _btab_body,
        out_shape=jax.ShapeDtypeStruct((N_PAD, FEATS), jnp.float32),
    )(ad, d0, d1)


def _final_body(q0_ref, q1_ref, b_ref, out_ref):
    out_ref[...] = q0_ref[...] + q1_ref[...] + b_ref[...]


def _tc_final(q0, q1, b):
    return pl.pallas_call(
        _final_body,
        out_shape=jax.ShapeDtypeStruct((N_PAD, FEATS), jnp.float32),
    )(q0, q1, b)


def _leaky_exp(a):
    return jnp.exp(jnp.where(a > 0, a, 0.2 * a))


def _zero_accumulator(buf_v, rows, acc_sh, sid):
    @pl.loop(0, rows)
    def _z(j):
        for c in range(FEATS // L):
            buf_v[j, pl.ds(c * L, L)] = jnp.zeros((L,), jnp.float32)

    @pl.loop(0, ROWS_SC // L)
    def _init(r):
        pltpu.sync_copy(buf_v.at[pl.ds(0, L)],
                        acc_sh.at[pl.ds(sid * ROWS_SC + r * L, L)])


@functools.partial(
    pl.kernel,
    mesh=_mesh,
    out_type=(jax.ShapeDtypeStruct((N_PAD, FEATS), jnp.float32),
              jax.ShapeDtypeStruct((N_PAD, FEATS), jnp.float32)),
    scratch_types=[pltpu.VMEM((SUP1,), jnp.int32),
                   pltpu.VMEM((SUP1,), jnp.int32),
                   pltpu.VMEM((K1,), jnp.int32),
                   pltpu.VMEM((K1,), jnp.int32),
                   pltpu.VMEM((K1, FEATS), jnp.float32),
                   pltpu.VMEM((K1, FEATS), jnp.float32),
                   pltpu.VMEM((K1, FEATS), jnp.float32),
                   pltpu.VMEM((K1, FEATS), jnp.float32),
                   pltpu.VMEM((K1, FEATS), jnp.float32),
                   pltpu.VMEM((K1, FEATS), jnp.float32),
                   pltpu.VMEM_SHARED((N_PAD, FEATS), jnp.float32),
                   pltpu.SemaphoreType.DMA,
                   pltpu.SemaphoreType.DMA,
                   pltpu.SemaphoreType.DMA,
                   pltpu.SemaphoreType.DMA,
                   pltpu.SemaphoreType.DMA,
                   pltpu.SemaphoreType.DMA],
)
def _sc_pass1(src_hbm, dst_hbm, as_hbm, ad_hbm,
              d0_hbm, d1_hbm,
              sa_v, da_v, dv0, dv1, as0_v, ad0_v, as1_v, ad1_v, ex0_v, ex1_v,
              den_sh,
              sema0, semd0, sema1, semd1, sems0, sems1):
    cid = lax.axis_index("c")
    sid = lax.axis_index("s")
    wid = sid * NC + cid

    _zero_accumulator(ex0_v, K1, den_sh, sid)
    plsc.subcore_barrier()

    base = wid * EP_TILE

    def _issue(c, asb, adb, sema, semd):
        pltpu.async_copy(as_hbm.at[sa_v.at[pl.ds(c * K1, K1)]], asb, sema)
        pltpu.async_copy(ad_hbm.at[da_v.at[pl.ds(c * K1, K1)]], adb, semd)

    def _process(s, c, asb, adb, exb, dvb, sema, semd, sems):
        pltpu.make_async_copy(as_hbm.at[sa_v.at[pl.ds(c * K1, K1)]],
                              asb, sema).wait()
        pltpu.make_async_copy(ad_hbm.at[da_v.at[pl.ds(c * K1, K1)]],
                              adb, semd).wait()

        g0 = s * CPS1 + c

        @pl.when(g0 >= 2)
        def _():
            pltpu.make_async_copy(exb, den_sh.at[dvb], sems).wait()

        @plsc.parallel_loop(0, K1, unroll=2)
        def _edge(j):
            exb[j, pl.ds(0, L)] = _leaky_exp(asb[j, pl.ds(0, L)]
                                             + adb[j, pl.ds(0, L)])

        @pl.loop(0, K1 // L)
        def _cdv(j):
            dvb[pl.ds(j * L, L)] = da_v[pl.ds(c * K1 + j * L, L)]

        pltpu.async_copy(exb, den_sh.at[dvb], sems, add=True)

    @pl.loop(0, NSUP1)
    def _super(s):
        soff = base + s * SUP1
        pltpu.sync_copy(src_hbm.at[pl.ds(soff, SUP1)], sa_v)
        pltpu.sync_copy(dst_hbm.at[pl.ds(soff, SUP1)], da_v)
        _issue(0, as0_v, ad0_v, sema0, semd0)

        @pl.loop(0, CPS1 // 2)
        def _pair(p):
            c0 = 2 * p
            _issue(c0 + 1, as1_v, ad1_v, sema1, semd1)
            _process(s, c0, as0_v, ad0_v, ex0_v, dv0, sema0, semd0, sems0)

            @pl.when(c0 + 2 < CPS1)
            def _():
                _issue(c0 + 2, as0_v, ad0_v, sema0, semd0)

            _process(s, c0 + 1, as1_v, ad1_v, ex1_v, dv1, sema1, semd1,
                     sems1)

    pltpu.make_async_copy(ex0_v, den_sh.at[dv0], sems0).wait()
    pltpu.make_async_copy(ex1_v, den_sh.at[dv1], sems1).wait()
    plsc.subcore_barrier()

    stripe = pl.ds(sid * ROWS_SC, ROWS_SC)

    @pl.when(cid == 0)
    def _():
        pltpu.sync_copy(den_sh.at[stripe], d0_hbm.at[stripe])

    @pl.when(cid == 1)
    def _():
        pltpu.sync_copy(den_sh.at[stripe], d1_hbm.at[stripe])


@functools.partial(
    pl.kernel,
    mesh=_mesh,
    compiler_params=_sc_params,
    out_type=(jax.ShapeDtypeStruct((N_PAD, FEATS), jnp.float32),
              jax.ShapeDtypeStruct((N_PAD, FEATS), jnp.float32)),
    scratch_types=[pltpu.VMEM((SUP2,), jnp.int32),
                   pltpu.VMEM((SUP2,), jnp.int32),
                   pltpu.VMEM((K2,), jnp.int32),
                   pltpu.VMEM((K2,), jnp.int32),
                   pltpu.VMEM((K2, XW), jnp.float32),
                   pltpu.VMEM((K2, XW), jnp.float32),
                   pltpu.VMEM((K2, FEATS), jnp.float32),
                   pltpu.VMEM((K2, FEATS), jnp.float32),
                   pltpu.VMEM((K2, FEATS), jnp.float32),
                   pltpu.VMEM((K2, FEATS), jnp.float32),
                   pltpu.VMEM_SHARED((N_PAD, FEATS), jnp.float32),
                   pltpu.SemaphoreType.DMA,
                   pltpu.SemaphoreType.DMA,
                   pltpu.SemaphoreType.DMA,
                   pltpu.SemaphoreType.DMA,
                   pltpu.SemaphoreType.DMA,
                   pltpu.SemaphoreType.DMA],
)
def _sc_pass2(src_hbm, dst_hbm, xh_hbm, b_hbm,
              p0_hbm, p1_hbm,
              sa_v, da_v, dv0, dv1, x0_v, x1_v, b0_v, b1_v, m0_v, m1_v,
              acc_sh,
              semb0, semx0, semb1, semx1, sems0, sems1):
    cid = lax.axis_index("c")
    sid = lax.axis_index("s")
    wid = sid * NC + cid

    _zero_accumulator(m0_v, K2, acc_sh, sid)
    plsc.subcore_barrier()

    base = wid * EP_TILE

    def _issue(c, xbuf, bbuf, semb, semx):
        pltpu.async_copy(b_hbm.at[da_v.at[pl.ds(c * K2, K2)]], bbuf, semb)
        pltpu.async_copy(xh_hbm.at[sa_v.at[pl.ds(c * K2, K2)]], xbuf, semx)

    def _process(s, c, xbuf, bbuf, mbuf, dvbuf, semb, semx, sems):
        pltpu.make_async_copy(b_hbm.at[da_v.at[pl.ds(c * K2, K2)]],
                              bbuf, semb).wait()
        pltpu.make_async_copy(xh_hbm.at[sa_v.at[pl.ds(c * K2, K2)]],
                              xbuf, semx).wait()

        g0 = s * CPS2 + c

        @pl.when(g0 >= 2)
        def _():
            pltpu.make_async_copy(mbuf, acc_sh.at[dvbuf], sems).wait()

        @plsc.parallel_loop(0, K2, unroll=2)
        def _edge(j):
            a = xbuf[j, pl.ds(HC, L)] + bbuf[j, pl.ds(0, L)]
            w_row = _leaky_exp(a) * bbuf[j, pl.ds(L, L)] * (1.0 / HEADS)
            wb = [jnp.broadcast_to(w_row[h], (L,)) for h in range(HEADS)]
            for g in range(FEATS // L):
                acc = wb[0] * xbuf[j, pl.ds(g * L, L)]
                for h in range(1, HEADS):
                    acc = acc + wb[h] * xbuf[j, pl.ds(h * FEATS + g * L, L)]
                mbuf[j, pl.ds(g * L, L)] = acc

        dvbuf[pl.ds(0, K2)] = da_v[pl.ds(c * K2, K2)]
        pltpu.async_copy(mbuf, acc_sh.at[dvbuf], sems, add=True)

    @pl.loop(0, NSUP2)
    def _super(s):
        soff = base + s * SUP2
        pltpu.sync_copy(src_hbm.at[pl.ds(soff, SUP2)], sa_v)
        pltpu.sync_copy(dst_hbm.at[pl.ds(soff, SUP2)], da_v)
        _issue(0, x0_v, b0_v, semb0, semx0)

        @pl.loop(0, CPS2 // 2)
        def _pair(p):
            c0 = 2 * p
            _issue(c0 + 1, x1_v, b1_v, semb1, semx1)
            _process(s, c0, x0_v, b0_v, m0_v, dv0, semb0, semx0, sems0)

            @pl.when(c0 + 2 < CPS2)
            def _():
                _issue(c0 + 2, x0_v, b0_v, semb0, semx0)

            _process(s, c0 + 1, x1_v, b1_v, m1_v, dv1, semb1, semx1, sems1)

    pltpu.make_async_copy(m0_v, acc_sh.at[dv0], sems0).wait()
    pltpu.make_async_copy(m1_v, acc_sh.at[dv1], sems1).wait()
    plsc.subcore_barrier()

    stripe = pl.ds(sid * ROWS_SC, ROWS_SC)

    @pl.when(cid == 0)
    def _():
        pltpu.sync_copy(acc_sh.at[stripe], p0_hbm.at[stripe])

    @pl.when(cid == 1)
    def _():
        pltpu.sync_copy(acc_sh.at[stripe], p1_hbm.at[stripe])


def kernel(x, edge_index, W1, att_src1, att_dst1, b1, W2, att_src2, att_dst2, b2):
    ei = edge_index.astype(jnp.int32)
    loops = jnp.arange(N_NODES, dtype=jnp.int32)
    src = jnp.concatenate([ei[0], loops])
    dst = jnp.concatenate([ei[1], loops])
    src = jnp.pad(src, (0, E_PAD - E_TOT))
    dst = jnp.pad(dst, (0, E_PAD - E_TOT), constant_values=DUMMY)
    x_p = jnp.pad(x, ((0, N_PAD - N_NODES), (0, 0)))
    b1r = b1.reshape(1, FEATS)
    b2r = b2.reshape(1, FEATS)

    xh1, s1, t1 = _tc_prep(x_p, W1, att_src1, att_dst1)
    d0, d1 = _sc_pass1(src, dst, s1, t1)
    btab1 = _tc_btab(t1, d0, d1)
    p0, p1 = _sc_pass2(src, dst, xh1, btab1)
    _, xh2, s2, t2 = _tc_relu_prep(p0, p1, b1r, W2, att_src2, att_dst2)
    e0, e1 = _sc_pass1(src, dst, s2, t2)
    btab2 = _tc_btab(t2, e0, e1)
    q0, q1 = _sc_pass2(src, dst, xh2, btab2)
    out = _tc_final(q0, q1, b2r)
    return out[:N_NODES, :]
